# Initial kernel scaffold; baseline (speedup 1.0000x reference)
import jax
import jax.numpy as jnp
from jax.experimental import pallas as pl


def kernel(features0, features1, features2, vertices0, vertices1, vertices2):
    raise NotImplementedError("write your pallas kernel here")


# speedup vs baseline: 1.9396x; 1.9396x over previous
import functools

import jax
import jax.numpy as jnp
from jax import lax
from jax.experimental import pallas as pl
from jax.experimental.pallas import tpu as pltpu
from jax.experimental.pallas import tpu_sc as plsc

C = 16
V = 262144
NW = 32
PTS_PER_W = V // NW
CHUNK = 128
N_CHUNKS = PTS_PER_W // CHUNK
L = 16
GROUPS = CHUNK // L

DIMS = ((128, 128, 16), (128, 16, 128), (16, 128, 128))


def _norm_tc_body(v_ref, t_ref):
    x = v_ref[...]
    mn = jnp.min(x)
    mx = jnp.max(x)
    t_ref[...] = (x - mn) / (mx - mn)


def _normalize01(v9):
    vb = v9.reshape(9, 2048, 128)
    t = pl.pallas_call(
        _norm_tc_body,
        grid=(9,),
        in_specs=[pl.BlockSpec((1, 2048, 128), lambda r: (r, 0, 0))],
        out_specs=pl.BlockSpec((1, 2048, 128), lambda r: (r, 0, 0)),
        out_shape=jax.ShapeDtypeStruct((9, 2048, 128), jnp.float32),
    )(vb)
    return t.reshape(9, V)


def _sc_body(t9, tab0, tab1, tab2, f0, f1, f2, out0, out1, out2,
             tv, idxb, wb, rows, accv, sem):
    wid = lax.axis_index("s") * 2 + lax.axis_index("c")
    tabs = (tab0, tab1, tab2)
    fs = (f0, f1, f2)
    outs = (out0, out1, out2)
    lane = lax.iota(jnp.int32, 16)

    for i in range(3):
        srcs = [j for j in range(3) if j != i]

        def chunk_body(ci, _, i=i, srcs=srcs):
            base = wid * PTS_PER_W + ci * CHUNK
            for d in range(3):
                pltpu.sync_copy(t9.at[pl.ds((3 * i + d) * V + base, CHUNK)],
                                tv.at[d])
            pltpu.sync_copy(fs[i].at[:, pl.ds(base, CHUNK)], accv)

            for j in srcs:
                D, H, W = DIMS[j]
                offs = tuple(dz * (H * W) + dy * W + dx
                             for dz in (0, 1) for dy in (0, 1)
                             for dx in (0, 1))

                def group_a(g, _, D=D, H=H, W=W, offs=offs):
                    s = g * L
                    tx = tv[0, pl.ds(s, L)]
                    ty = tv[1, pl.ds(s, L)]
                    tz = tv[2, pl.ds(s, L)]
                    x = tx * jnp.float32(W - 1)
                    y = ty * jnp.float32(H - 1)
                    z = tz * jnp.float32(D - 1)
                    x0 = jnp.minimum(jnp.maximum(x.astype(jnp.int32), 0),
                                     W - 2)
                    y0 = jnp.minimum(jnp.maximum(y.astype(jnp.int32), 0),
                                     H - 2)
                    z0 = jnp.minimum(jnp.maximum(z.astype(jnp.int32), 0),
                                     D - 2)
                    wx1 = x - x0.astype(jnp.float32)
                    wy1 = y - y0.astype(jnp.float32)
                    wz1 = z - z0.astype(jnp.float32)
                    wx0 = 1.0 - wx1
                    wy0 = 1.0 - wy1
                    wz0 = 1.0 - wz1
                    v00 = (z0 * H + y0) * W + x0
                    wz_ = (wz0, wz1)
                    wy_ = (wy0, wy1)
                    wx_ = (wx0, wx1)
                    cidx = 0
                    for dz in (0, 1):
                        for dy in (0, 1):
                            zy = wz_[dz] * wy_[dy]
                            for dx in (0, 1):
                                idxb[cidx, pl.ds(s, L)] = v00 + offs[cidx]
                                wb[cidx, pl.ds(s, L)] = zy * wx_[dx]
                                cidx += 1
                    return 0

                lax.fori_loop(0, GROUPS, group_a, 0)

                cps = [pltpu.async_copy(tabs[j].at[idxb.at[c]], rows.at[c],
                                        sem)
                       for c in range(8)]
                for cp in cps:
                    cp.wait()

                def group_b(g, _):
                    s = g * L
                    pts = lane + s
                    wvecs = [wb[c, pl.ds(s, L)] for c in range(8)]
                    cvecs = [jnp.full((L,), c, jnp.int32) for c in range(8)]
                    for ch in range(C):
                        acc = accv[ch, pl.ds(s, L)]
                        chv = jnp.full((L,), ch, jnp.int32)
                        for c in range(8):
                            rv = plsc.load_gather(rows,
                                                  [cvecs[c], pts, chv])
                            acc = acc + wvecs[c] * rv
                        accv[ch, pl.ds(s, L)] = acc
                    return 0

                lax.fori_loop(0, GROUPS, group_b, 0)

            pltpu.sync_copy(accv, outs[i].at[:, pl.ds(base, CHUNK)])
            return 0

        lax.fori_loop(0, N_CHUNKS, chunk_body, 0)


@functools.partial(jax.jit, static_argnames=("interpret",))
def _run(feats, verts, interpret=False):
    v9 = jnp.stack([v.reshape(V, 3).T for v in verts]).reshape(9, V)
    t9 = _normalize01(v9)
    tabs = [f.reshape(C, V).T for f in feats]
    fs = [f.reshape(C, V) for f in feats]

    mesh = plsc.VectorSubcoreMesh(core_axis_name="c", subcore_axis_name="s",
                                  num_cores=2, num_subcores=16)
    outs = pl.kernel(
        _sc_body,
        out_type=[jax.ShapeDtypeStruct((C, V), jnp.float32)] * 3,
        mesh=mesh,
        scratch_types=[
            pltpu.VMEM((3, CHUNK), jnp.float32),
            pltpu.VMEM((8, CHUNK), jnp.int32),
            pltpu.VMEM((8, CHUNK), jnp.float32),
            pltpu.VMEM((8, CHUNK, C), jnp.float32),
            pltpu.VMEM((C, CHUNK), jnp.float32),
            pltpu.SemaphoreType.DMA,
        ],
        compiler_params=pltpu.CompilerParams(needs_layout_passes=False,
                                             use_tc_tiling_on_sc=False),
        interpret=interpret,
    )(t9.reshape(-1), *tabs, *fs)
    return outs


def kernel(features0, features1, features2,
           vertices0, vertices1, vertices2):
    feats = (features0, features1, features2)
    verts = (vertices0, vertices1, vertices2)
    outs = _run(feats, verts)
    return tuple(o.reshape(f.shape) for o, f in zip(outs, feats))


# speedup vs baseline: 2.7254x; 1.4052x over previous
import functools

import jax
import jax.numpy as jnp
from jax import lax
from jax.experimental import pallas as pl
from jax.experimental.pallas import tpu as pltpu
from jax.experimental.pallas import tpu_sc as plsc

C = 16
V = 262144
NW = 32
PTS_PER_W = V // NW
CHUNK = 128
N_CHUNKS = PTS_PER_W // CHUNK
L = 16
GROUPS = CHUNK // L

DIMS = ((128, 128, 16), (128, 16, 128), (16, 128, 128))

SEED_BYTES = C * CHUNK * 4
GATHER_BYTES = 16 * CHUNK * C * 4 + SEED_BYTES
OUT_BYTES = C * CHUNK * 4


def _norm_tc_body(v_ref, t_ref):
    x = v_ref[...]
    mn = jnp.min(x)
    mx = jnp.max(x)
    t_ref[...] = (x - mn) / (mx - mn)


def _normalize01(v9):
    vb = v9.reshape(9, 2048, 128)
    t = pl.pallas_call(
        _norm_tc_body,
        grid=(9,),
        in_specs=[pl.BlockSpec((1, 2048, 128), lambda r: (r, 0, 0))],
        out_specs=pl.BlockSpec((1, 2048, 128), lambda r: (r, 0, 0)),
        out_shape=jax.ShapeDtypeStruct((9, 2048, 128), jnp.float32),
    )(vb)
    return t.reshape(-1)


def _sc_body(t9, tab0, tab1, tab2, f0, f1, f2, out0, out1, out2,
             tv, idxb0, idxb1, wb0, wb1, rows0, rows1, acc0, acc1,
             semg0, semg1, semo0, semo1):
    wid = lax.axis_index("s") * 2 + lax.axis_index("c")
    tabs = (tab0, tab1, tab2)
    fs = (f0, f1, f2)
    outs = (out0, out1, out2)
    idxbs = (idxb0, idxb1)
    wbs = (wb0, wb1)
    rowss = (rows0, rows1)
    accs = (acc0, acc1)
    semgs = (semg0, semg1)
    semos = (semo0, semo1)
    lane = lax.iota(jnp.int32, 16)
    cvecs = [jnp.full((L,), c, jnp.int32) for c in range(16)]

    for i in range(3):
        srcs = [j for j in range(3) if j != i]

        for d in range(3):
            pltpu.sync_copy(
                t9.at[pl.ds((3 * i + d) * V + wid * PTS_PER_W, PTS_PER_W)],
                tv.at[d])

        def stage(ci, b, w, i=i, srcs=srcs):
            base = wid * PTS_PER_W + ci * CHUNK
            coff = ci * CHUNK
            def _wait_out():
                pltpu.make_async_copy(
                    accs[b], outs[i].at[:, pl.ds(0, CHUNK)],
                    semos[b]).wait()

            if w is True:
                _wait_out()
            elif w is not None:
                pl.when(w)(_wait_out)
            pltpu.async_copy(fs[i].at[:, pl.ds(base, CHUNK)], accs[b],
                             semgs[b])

            def ga(g, _):
                s = g * L
                tx = tv[0, pl.ds(coff + s, L)]
                ty = tv[1, pl.ds(coff + s, L)]
                tz = tv[2, pl.ds(coff + s, L)]
                for jp, j in enumerate(srcs):
                    D, H, W = DIMS[j]
                    x = tx * jnp.float32(W - 1)
                    y = ty * jnp.float32(H - 1)
                    z = tz * jnp.float32(D - 1)
                    x0 = jnp.minimum(
                        jnp.maximum(x.astype(jnp.int32), 0), W - 2)
                    y0 = jnp.minimum(
                        jnp.maximum(y.astype(jnp.int32), 0), H - 2)
                    z0 = jnp.minimum(
                        jnp.maximum(z.astype(jnp.int32), 0), D - 2)
                    wx1 = x - x0.astype(jnp.float32)
                    wy1 = y - y0.astype(jnp.float32)
                    wz1 = z - z0.astype(jnp.float32)
                    wz_ = (1.0 - wz1, wz1)
                    wy_ = (1.0 - wy1, wy1)
                    wx_ = (1.0 - wx1, wx1)
                    v00 = (z0 * H + y0) * W + x0
                    cix = 0
                    for dz in (0, 1):
                        for dy in (0, 1):
                            zy = wz_[dz] * wy_[dy]
                            for dx in (0, 1):
                                off = dz * (H * W) + dy * W + dx
                                cg = jp * 8 + cix
                                idxbs[b][cg, pl.ds(s, L)] = v00 + off
                                wbs[b][cg, pl.ds(s, L)] = zy * wx_[dx]
                                cix += 1
                return 0

            lax.fori_loop(0, GROUPS, ga, 0)

            for jp, j in enumerate(srcs):
                for c in range(8):
                    cg = jp * 8 + c
                    pltpu.async_copy(tabs[j].at[idxbs[b].at[cg]],
                                     rowss[b].at[cg], semgs[b])

        def compute(ci, b, i=i):
            base = wid * PTS_PER_W + ci * CHUNK
            pltpu.make_async_copy(fs[i].at[:, pl.ds(base, CHUNK)], accs[b],
                                  semgs[b]).wait()
            for cg in range(16):
                pltpu.make_async_copy(tabs[0].at[idxbs[b].at[cg]],
                                      rowss[b].at[cg], semgs[b]).wait()

            def gb(g, _):
                s = g * L
                pts = lane + s
                wv = [wbs[b][c, pl.ds(s, L)] for c in range(16)]
                for ch in range(C):
                    acc = accs[b][ch, pl.ds(s, L)]
                    chv = jnp.full((L,), ch, jnp.int32)
                    for c in range(16):
                        rv = plsc.load_gather(rowss[b],
                                              [cvecs[c], pts, chv])
                        acc = acc + wv[c] * rv
                    accs[b][ch, pl.ds(s, L)] = acc
                return 0

            lax.fori_loop(0, GROUPS, gb, 0)
            pltpu.async_copy(accs[b], outs[i].at[:, pl.ds(base, CHUNK)],
                             semos[b])

        stage(jnp.int32(0), 0, True if i > 0 else None)

        def step(k, _, stage=stage, compute=compute, i=i):
            ci0 = k * 2
            stage(ci0 + 1, 1, True if i > 0 else k >= 1)
            compute(ci0, 0)

            @pl.when(k < N_CHUNKS // 2 - 1)
            def _():
                stage(ci0 + 2, 0, True)

            compute(ci0 + 1, 1)
            return 0

        lax.fori_loop(0, N_CHUNKS // 2, step, 0)

    pltpu.make_async_copy(acc0, out2.at[:, pl.ds(0, CHUNK)], semo0).wait()
    pltpu.make_async_copy(acc1, out2.at[:, pl.ds(0, CHUNK)], semo1).wait()


@functools.partial(jax.jit, static_argnames=("interpret",))
def _run(feats, verts, interpret=False):
    v9 = jnp.stack([v.reshape(V, 3).T for v in verts]).reshape(9, V)
    t9 = _normalize01(v9)
    tabs = [f.reshape(C, V).T for f in feats]
    fs = [f.reshape(C, V) for f in feats]

    mesh = plsc.VectorSubcoreMesh(core_axis_name="c", subcore_axis_name="s",
                                  num_cores=2, num_subcores=16)
    outs = pl.kernel(
        _sc_body,
        out_type=[jax.ShapeDtypeStruct((C, V), jnp.float32)] * 3,
        mesh=mesh,
        scratch_types=[
            pltpu.VMEM((3, PTS_PER_W), jnp.float32),
            pltpu.VMEM((16, CHUNK), jnp.int32),
            pltpu.VMEM((16, CHUNK), jnp.int32),
            pltpu.VMEM((16, CHUNK), jnp.float32),
            pltpu.VMEM((16, CHUNK), jnp.float32),
            pltpu.VMEM((16, CHUNK, C), jnp.float32),
            pltpu.VMEM((16, CHUNK, C), jnp.float32),
            pltpu.VMEM((C, CHUNK), jnp.float32),
            pltpu.VMEM((C, CHUNK), jnp.float32),
            pltpu.SemaphoreType.DMA,
            pltpu.SemaphoreType.DMA,
            pltpu.SemaphoreType.DMA,
            pltpu.SemaphoreType.DMA,
        ],
        compiler_params=pltpu.CompilerParams(needs_layout_passes=False,
                                             use_tc_tiling_on_sc=False),
        interpret=interpret,
    )(t9, *tabs, *fs)
    return outs


def kernel(features0, features1, features2,
           vertices0, vertices1, vertices2):
    feats = (features0, features1, features2)
    verts = (vertices0, vertices1, vertices2)
    outs = _run(feats, verts)
    return tuple(o.reshape(f.shape) for o, f in zip(outs, feats))


# speedup vs baseline: 6.1383x; 2.2523x over previous
import functools

import jax
import jax.numpy as jnp
from jax import lax
from jax.experimental import pallas as pl
from jax.experimental.pallas import tpu as pltpu
from jax.experimental.pallas import tpu_sc as plsc

C = 16
V = 262144
NW = 32
PTS_PER_W = V // NW
CHUNK = 128
N_CHUNKS = PTS_PER_W // CHUNK
L = 16
GROUPS = CHUNK // L

DIMS = ((128, 128, 16), (128, 16, 128), (16, 128, 128))

SEED_BYTES = C * CHUNK * 4
GATHER_BYTES = 16 * CHUNK * C * 4 + SEED_BYTES
OUT_BYTES = C * CHUNK * 4


def _norm_tc_body(v_ref, t_ref):
    x = v_ref[...]
    mn = jnp.min(x)
    mx = jnp.max(x)
    t_ref[...] = (x - mn) / (mx - mn)


def _normalize01(v9):
    vb = v9.reshape(9, 2048, 128)
    t = pl.pallas_call(
        _norm_tc_body,
        grid=(9,),
        in_specs=[pl.BlockSpec((1, 2048, 128), lambda r: (r, 0, 0))],
        out_specs=pl.BlockSpec((1, 2048, 128), lambda r: (r, 0, 0)),
        out_shape=jax.ShapeDtypeStruct((9, 2048, 128), jnp.float32),
    )(vb)
    return t.reshape(-1)


def _sc_body(t9, tab0, tab1, tab2, out0, out1, out2,
             tv, idxb0, idxb1, wb0, wb1, rows0, rows1, acc0, acc1,
             semg0, semg1, semo0, semo1):
    wid = lax.axis_index("s") * 2 + lax.axis_index("c")
    tabs = (tab0, tab1, tab2)
    outs = (out0, out1, out2)
    idxbs = (idxb0, idxb1)
    wbs = (wb0, wb1)
    rowss = (rows0, rows1)
    accs = (acc0, acc1)
    semgs = (semg0, semg1)
    semos = (semo0, semo1)
    lane = lax.iota(jnp.int32, 16)
    cvecs = [jnp.full((L,), c, jnp.int32) for c in range(16)]

    for i in range(3):
        srcs = [j for j in range(3) if j != i]

        for d in range(3):
            pltpu.sync_copy(
                t9.at[pl.ds((3 * i + d) * V + wid * PTS_PER_W, PTS_PER_W)],
                tv.at[d])

        def stage(ci, b, w, i=i, srcs=srcs):
            base = wid * PTS_PER_W + ci * CHUNK
            coff = ci * CHUNK
            def _wait_out():
                pltpu.make_async_copy(
                    accs[b], outs[i].at[pl.ds(0, CHUNK)], semos[b]).wait()

            if w is True:
                _wait_out()
            elif w is not None:
                pl.when(w)(_wait_out)
            pltpu.async_copy(tabs[i].at[pl.ds(base, CHUNK)], accs[b],
                             semgs[b])

            def ga(g, _):
                s = g * L
                tx = tv[0, pl.ds(coff + s, L)]
                ty = tv[1, pl.ds(coff + s, L)]
                tz = tv[2, pl.ds(coff + s, L)]
                for jp, j in enumerate(srcs):
                    D, H, W = DIMS[j]
                    x = tx * jnp.float32(W - 1)
                    y = ty * jnp.float32(H - 1)
                    z = tz * jnp.float32(D - 1)
                    x0 = jnp.minimum(
                        jnp.maximum(x.astype(jnp.int32), 0), W - 2)
                    y0 = jnp.minimum(
                        jnp.maximum(y.astype(jnp.int32), 0), H - 2)
                    z0 = jnp.minimum(
                        jnp.maximum(z.astype(jnp.int32), 0), D - 2)
                    wx1 = x - x0.astype(jnp.float32)
                    wy1 = y - y0.astype(jnp.float32)
                    wz1 = z - z0.astype(jnp.float32)
                    wz_ = (1.0 - wz1, wz1)
                    wy_ = (1.0 - wy1, wy1)
                    wx_ = (1.0 - wx1, wx1)
                    v00 = (z0 * H + y0) * W + x0
                    cix = 0
                    for dz in (0, 1):
                        for dy in (0, 1):
                            zy = wz_[dz] * wy_[dy]
                            for dx in (0, 1):
                                off = dz * (H * W) + dy * W + dx
                                cg = jp * 8 + cix
                                idxbs[b][cg, pl.ds(s, L)] = v00 + off
                                wbs[b][cg, pl.ds(s, L)] = zy * wx_[dx]
                                cix += 1
                return 0

            lax.fori_loop(0, GROUPS, ga, 0)

            for jp, j in enumerate(srcs):
                for c in range(8):
                    cg = jp * 8 + c
                    pltpu.async_copy(tabs[j].at[idxbs[b].at[cg]],
                                     rowss[b].at[cg], semgs[b])

        def compute(ci, b, i=i):
            base = wid * PTS_PER_W + ci * CHUNK
            pltpu.make_async_copy(tabs[i].at[pl.ds(base, CHUNK)], accs[b],
                                  semgs[b]).wait()
            for cg in range(16):
                pltpu.make_async_copy(tabs[0].at[idxbs[b].at[cg]],
                                      rowss[b].at[cg], semgs[b]).wait()

            def gb(g, _):
                s = g * L
                wv = [wbs[b][c, pl.ds(s, L)] for c in range(16)]
                for l in range(L):
                    p = s + l
                    lsp = jnp.full((L,), l, jnp.int32)
                    acc = accs[b][p, :]
                    for cg in range(16):
                        rv = rowss[b][cg, p, :]
                        wsp = wv[cg].at[lsp].get(mode="promise_in_bounds")
                        acc = acc + wsp * rv
                    accs[b][p, :] = acc
                return 0

            lax.fori_loop(0, GROUPS, gb, 0)
            pltpu.async_copy(accs[b], outs[i].at[pl.ds(base, CHUNK)],
                             semos[b])

        stage(jnp.int32(0), 0, True if i > 0 else None)

        def step(k, _, stage=stage, compute=compute, i=i):
            ci0 = k * 2
            stage(ci0 + 1, 1, True if i > 0 else k >= 1)
            compute(ci0, 0)

            @pl.when(k < N_CHUNKS // 2 - 1)
            def _():
                stage(ci0 + 2, 0, True)

            compute(ci0 + 1, 1)
            return 0

        lax.fori_loop(0, N_CHUNKS // 2, step, 0)

    pltpu.make_async_copy(acc0, out2.at[pl.ds(0, CHUNK)], semo0).wait()
    pltpu.make_async_copy(acc1, out2.at[pl.ds(0, CHUNK)], semo1).wait()


@functools.partial(jax.jit, static_argnames=("interpret",))
def _run(feats, verts, interpret=False):
    v9 = jnp.stack([v.reshape(V, 3).T for v in verts]).reshape(9, V)
    t9 = _normalize01(v9)
    tabs = [f.reshape(C, V).T for f in feats]

    mesh = plsc.VectorSubcoreMesh(core_axis_name="c", subcore_axis_name="s",
                                  num_cores=2, num_subcores=16)
    outs = pl.kernel(
        _sc_body,
        out_type=[jax.ShapeDtypeStruct((V, C), jnp.float32)] * 3,
        mesh=mesh,
        scratch_types=[
            pltpu.VMEM((3, PTS_PER_W), jnp.float32),
            pltpu.VMEM((16, CHUNK), jnp.int32),
            pltpu.VMEM((16, CHUNK), jnp.int32),
            pltpu.VMEM((16, CHUNK), jnp.float32),
            pltpu.VMEM((16, CHUNK), jnp.float32),
            pltpu.VMEM((16, CHUNK, C), jnp.float32),
            pltpu.VMEM((16, CHUNK, C), jnp.float32),
            pltpu.VMEM((CHUNK, C), jnp.float32),
            pltpu.VMEM((CHUNK, C), jnp.float32),
            pltpu.SemaphoreType.DMA,
            pltpu.SemaphoreType.DMA,
            pltpu.SemaphoreType.DMA,
            pltpu.SemaphoreType.DMA,
        ],
        compiler_params=pltpu.CompilerParams(needs_layout_passes=False,
                                             use_tc_tiling_on_sc=False),
        interpret=interpret,
    )(t9, *tabs)
    return outs


def kernel(features0, features1, features2,
           vertices0, vertices1, vertices2):
    feats = (features0, features1, features2)
    verts = (vertices0, vertices1, vertices2)
    outs = _run(feats, verts)
    return tuple(o.T.reshape(f.shape) for o, f in zip(outs, feats))
